# baseline (speedup 1.0000x reference)
import jax
import jax.numpy as jnp
from jax.experimental import pallas as pl
from jax.experimental.pallas import tpu as pltpu

_BF = jnp.bfloat16
_F32 = jnp.float32


def _dot(a, b):
    return jnp.dot(a, b, preferred_element_type=_F32)


def _conv1d(x, w, b, stride=1, pad=1):
    y = jax.lax.conv_general_dilated(x, w, (stride,), [(pad, pad)],
                                     dimension_numbers=('NCH', 'OIH', 'NCH'))
    return y + b[None, :, None]


def _encoder(x, p):
    h = _conv1d(x, p['enc_w_in'], p['enc_b_in'])
    for i in range(3):
        r = jax.nn.relu(h)
        r = _conv1d(r, p['enc_r%d_w1' % i], p['enc_r%d_b1' % i])
        r = jax.nn.relu(r)
        r = _conv1d(r, p['enc_r%d_w2' % i], p['enc_r%d_b2' % i])
        h = h + r
        h = _conv1d(h, p['enc_d%d_w' % i], p['enc_d%d_b' % i], stride=2, pad=1)
    h = jax.nn.relu(h)
    return _conv1d(h, p['enc_w_out'], p['enc_b_out'])


def _vq_kernel(flat_ref, w_in, b_in, cbt, cc, cbh, cbm, cbl, w_out, b_out,
               qlat_ref, idx_ref, loss_ref):
    rows = 256
    z_e = _dot(flat_ref[:].astype(_BF), w_in[:]) + b_in[:]
    res = z_e
    qsum = jnp.zeros_like(z_e)
    idx_cols = []
    il = jax.lax.broadcasted_iota(jnp.int32, (rows, 256), 1)
    for i in range(4):
        rb = res.astype(_BF)
        rr = jnp.sum(res * res, axis=1, keepdims=True)
        bestd = jnp.full((rows, 1), jnp.inf, _F32)
        besti = jnp.zeros((rows, 1), jnp.int32)
        for c in range(4):
            mmc = _dot(rb, cbt[i, :, c * 256:(c + 1) * 256])
            dc = (rr - 2.0 * mmc) + cc[i:i + 1, c * 256:(c + 1) * 256]
            m = jnp.min(dc, axis=1, keepdims=True)
            cand = jnp.where(dc == m, il, jnp.full_like(il, 2 ** 30))
            ci = jnp.min(cand, axis=1, keepdims=True) + c * 256
            upd = m < bestd
            besti = jnp.where(upd, ci, besti)
            bestd = jnp.where(upd, m, bestd)
        quant = None
        for c in range(4):
            oh = (besti == (il + c * 256)).astype(_BF)
            qp = (_dot(oh, cbh[i, c * 256:(c + 1) * 256, :])
                  + _dot(oh, cbm[i, c * 256:(c + 1) * 256, :])
                  + _dot(oh, cbl[i, c * 256:(c + 1) * 256, :]))
            quant = qp if quant is None else quant + qp
        qsum = qsum + quant
        res = res - quant
        idx_cols.append(besti)
    idx_ref[:, :] = jnp.concatenate(idx_cols, axis=1)
    sse = jnp.sum((z_e - qsum) ** 2)
    loss_ref[0, :, :] = jnp.full((1, 128), sse, _F32)
    qlat_ref[:, :] = _dot(qsum.astype(_BF), w_out[:]) + b_out[:]


def _zero_pads(dst, t_len, ncols):
    z = jnp.zeros((1, ncols), _F32)
    dst[0:1, :] = z
    dst[t_len + 1:t_len + 2, :] = z


def _conv3(src, dst, t_len, w, b, relu=False, accum=False, ncols=256):
    tt = min(256, t_len)
    for it in range(t_len // tt):
        t0 = it * tt
        y = None
        for k in range(3):
            s = src[t0 + k:t0 + k + tt, :].astype(_BF)
            if relu:
                s = jnp.maximum(s, jnp.zeros_like(s))
            term = _dot(s, w[k])
            y = term if y is None else y + term
        y = y + b[:]
        if accum:
            y = y + dst[1 + t0:1 + t0 + tt, :]
        dst[1 + t0:1 + t0 + tt, :] = y
    _zero_pads(dst, t_len, ncols)


def _conv3_out(src, out, t_len, w, b, relu=False):
    tt = min(256, t_len)
    for it in range(t_len // tt):
        t0 = it * tt
        y = None
        for k in range(3):
            s = src[t0 + k:t0 + k + tt, :].astype(_BF)
            if relu:
                s = jnp.maximum(s, jnp.zeros_like(s))
            term = _dot(s, w[k])
            y = term if y is None else y + term
        out[t0:t0 + tt, :] = y + b[:]


def _up2conv3(src, dst, t_in, w, b, ncols=256):
    tt = min(128, t_in)
    for it in range(t_in // tt):
        t0 = it * tt
        s0 = src[t0:t0 + tt, :].astype(_BF)
        s1 = src[t0 + 1:t0 + 1 + tt, :].astype(_BF)
        s2 = src[t0 + 2:t0 + 2 + tt, :].astype(_BF)
        a = _dot(s0, w[0])
        bm = _dot(s1, w[0])
        c = _dot(s1, w[1])
        d = _dot(s1, w[2])
        e = _dot(s2, w[2])
        ye = (a + c) + d + b[:]
        yo = (bm + c) + e + b[:]
        inter = jnp.concatenate([ye, yo], axis=1).reshape(2 * tt, ncols)
        dst[1 + 2 * t0:1 + 2 * t0 + 2 * tt, :] = inter
    _zero_pads(dst, 2 * t_in, ncols)


def _dec_kernel(q_ref,
                wi, bi,
                u0w, u0b, r0w1, r0b1, r0w2, r0b2,
                u1w, u1b, r1w1, r1b1, r1w2, r1b2,
                u2w, u2b, r2w1, r2b1, r2w2, r2b2,
                wo, bo,
                out_ref,
                a, bbuf):
    a[1:65, :] = q_ref[0, :, :]
    _zero_pads(a, 64, 256)
    _conv3(a, bbuf, 64, wi, bi)
    _up2conv3(bbuf, a, 64, u0w, u0b)
    _conv3(a, bbuf, 128, r0w1, r0b1, relu=True)
    _conv3(bbuf, a, 128, r0w2, r0b2, relu=True, accum=True)
    _up2conv3(a, bbuf, 128, u1w, u1b)
    _conv3(bbuf, a, 256, r1w1, r1b1, relu=True)
    _conv3(a, bbuf, 256, r1w2, r1b2, relu=True, accum=True)
    _up2conv3(bbuf, a, 256, u2w, u2b)
    _conv3(a, bbuf, 512, r2w1, r2b1, relu=True)
    _conv3(bbuf, a, 512, r2w2, r2b2, relu=True, accum=True)
    _conv3_out(a, out_ref.at[0], 512, wo, bo, relu=True)


def _full(shape):
    nd = len(shape)
    return pl.BlockSpec(shape, lambda i, _nd=nd: (0,) * _nd)


def kernel(x, params):
    p = params
    bq = x.shape[0]

    feat = _encoder(jnp.transpose(x, (0, 2, 1)), p)
    flat = jnp.transpose(feat, (0, 2, 1)).reshape(-1, 256)
    nrows = flat.shape[0]

    cb = p['codebooks']
    msk = jnp.uint32(0xFFFF0000)
    hi = jax.lax.bitcast_convert_type(
        jax.lax.bitcast_convert_type(cb, jnp.uint32) & msk, _F32)
    rem = cb - hi
    mid = jax.lax.bitcast_convert_type(
        jax.lax.bitcast_convert_type(rem, jnp.uint32) & msk, _F32)
    cb_hi = hi.astype(_BF)
    cb_mid = mid.astype(_BF)
    cb_lo = (rem - mid).astype(_BF)
    cbt = jnp.transpose(cb, (0, 2, 1)).astype(_BF)
    cc = jnp.sum(cb * cb, axis=2)

    def b2(b):
        return b.reshape(1, -1)

    vq_in = [flat, p['vq_w_in'].astype(_BF), b2(p['vq_b_in']),
             cbt, cc, cb_hi, cb_mid, cb_lo,
             p['vq_w_out'].astype(_BF), b2(p['vq_b_out'])]
    vq_specs = [pl.BlockSpec((256, 256), lambda i: (i, 0))]
    vq_specs += [_full(a.shape) for a in vq_in[1:]]

    nblk = nrows // 256
    q_lat, enc_idx, loss_parts = pl.pallas_call(
        _vq_kernel,
        grid=(nblk,),
        in_specs=vq_specs,
        out_specs=[pl.BlockSpec((256, 256), lambda i: (i, 0)),
                   pl.BlockSpec((256, 4), lambda i: (i, 0)),
                   pl.BlockSpec((1, 1, 128), lambda i: (i, 0, 0))],
        out_shape=[jax.ShapeDtypeStruct((nrows, 256), _F32),
                   jax.ShapeDtypeStruct((nrows, 4), jnp.int32),
                   jax.ShapeDtypeStruct((nblk, 1, 128), _F32)],
    )(*vq_in)

    vq_loss = 0.25 * (jnp.sum(loss_parts[:, 0, 0]) / (nrows * 128))

    def tw(w):
        return jnp.transpose(w, (2, 1, 0)).astype(_BF)

    q = q_lat.reshape(bq, 64, 256)
    dec_in = [q]
    dec_specs = [pl.BlockSpec((1, 64, 256), lambda i: (i, 0, 0))]
    dec_in += [tw(p['dec_w_in']), b2(p['dec_b_in'])]
    for i in range(3):
        dec_in += [tw(p['dec_u%d_w' % i]), b2(p['dec_u%d_b' % i]),
                   tw(p['dec_r%d_w1' % i]), b2(p['dec_r%d_b1' % i]),
                   tw(p['dec_r%d_w2' % i]), b2(p['dec_r%d_b2' % i])]
    dec_in += [tw(p['dec_w_out']), b2(p['dec_b_out'])]
    dec_specs += [_full(a.shape) for a in dec_in[1:]]

    x_recon = pl.pallas_call(
        _dec_kernel,
        grid=(bq,),
        in_specs=dec_specs,
        out_specs=pl.BlockSpec((1, 512, 32), lambda i: (i, 0, 0)),
        out_shape=jax.ShapeDtypeStruct((bq, 512, 32), _F32),
        scratch_shapes=[pltpu.VMEM((520, 256), _F32),
                        pltpu.VMEM((520, 256), _F32)],
    )(*dec_in)

    return (x_recon, vq_loss, enc_idx)


# speedup vs baseline: 1.0214x; 1.0214x over previous
import jax
import jax.numpy as jnp
from jax.experimental import pallas as pl
from jax.experimental.pallas import tpu as pltpu

_BF = jnp.bfloat16
_F32 = jnp.float32


def _dot(a, b):
    return jnp.dot(a, b, preferred_element_type=_F32)


def _conv1d(x, w, b, stride=1, pad=1):
    y = jax.lax.conv_general_dilated(x, w, (stride,), [(pad, pad)],
                                     dimension_numbers=('NCH', 'OIH', 'NCH'))
    return y + b[None, :, None]


def _encoder(x, p):
    h = _conv1d(x, p['enc_w_in'], p['enc_b_in'])
    for i in range(3):
        r = jax.nn.relu(h)
        r = _conv1d(r, p['enc_r%d_w1' % i], p['enc_r%d_b1' % i])
        r = jax.nn.relu(r)
        r = _conv1d(r, p['enc_r%d_w2' % i], p['enc_r%d_b2' % i])
        h = h + r
        h = _conv1d(h, p['enc_d%d_w' % i], p['enc_d%d_b' % i], stride=2, pad=1)
    h = jax.nn.relu(h)
    return _conv1d(h, p['enc_w_out'], p['enc_b_out'])


def _vq_kernel(flat_ref, w_in, b_in, cbt, cc, cbh, cbm, cbl, w_out, b_out,
               qlat_ref, idx_ref, loss_ref):
    rows = 256
    z_e = _dot(flat_ref[:].astype(_BF), w_in[:]) + b_in[:]
    res = z_e
    qsum = jnp.zeros_like(z_e)
    idx_cols = []
    il = jax.lax.broadcasted_iota(jnp.int32, (rows, 256), 1)
    for i in range(4):
        rb = res.astype(_BF)
        rr = jnp.sum(res * res, axis=1, keepdims=True)
        bestd = jnp.full((rows, 1), jnp.inf, _F32)
        besti = jnp.zeros((rows, 1), jnp.int32)
        for c in range(4):
            mmc = _dot(rb, cbt[i, :, c * 256:(c + 1) * 256])
            dc = (rr - 2.0 * mmc) + cc[i:i + 1, c * 256:(c + 1) * 256]
            m = jnp.min(dc, axis=1, keepdims=True)
            cand = jnp.where(dc == m, il, jnp.full_like(il, 2 ** 30))
            ci = jnp.min(cand, axis=1, keepdims=True) + c * 256
            upd = m < bestd
            besti = jnp.where(upd, ci, besti)
            bestd = jnp.where(upd, m, bestd)
        quant = None
        for c in range(4):
            oh = (besti == (il + c * 256)).astype(_BF)
            qp = (_dot(oh, cbh[i, c * 256:(c + 1) * 256, :])
                  + _dot(oh, cbm[i, c * 256:(c + 1) * 256, :])
                  + _dot(oh, cbl[i, c * 256:(c + 1) * 256, :]))
            quant = qp if quant is None else quant + qp
        qsum = qsum + quant
        res = res - quant
        idx_cols.append(besti)
    idx_ref[:, :] = jnp.concatenate(idx_cols, axis=1)
    sse = jnp.sum((z_e - qsum) ** 2)
    loss_ref[0, :, :] = jnp.full((1, 128), sse, _F32)
    qlat_ref[:, :] = _dot(qsum.astype(_BF), w_out[:]) + b_out[:]


def _zero_pads(dst, t_len, ncols):
    z = jnp.zeros((1, ncols), _F32)
    dst[0:1, :] = z
    dst[t_len + 1:t_len + 2, :] = z


def _conv3(src, dst, t_len, w, b, relu=False, accum=False, ncols=256):
    tt = min(128, t_len)
    for it in range(t_len // tt):
        t0 = it * tt
        y = None
        for k in range(3):
            s = src[t0 + k:t0 + k + tt, :].astype(_BF)
            if relu:
                s = jnp.maximum(s, jnp.zeros_like(s))
            term = _dot(s, w[k])
            y = term if y is None else y + term
        y = y + b[:]
        if accum:
            y = y + dst[1 + t0:1 + t0 + tt, :]
        dst[1 + t0:1 + t0 + tt, :] = y
    _zero_pads(dst, t_len, ncols)


def _conv3_out(src, out, t_len, w, b, relu=False):
    tt = min(128, t_len)
    for it in range(t_len // tt):
        t0 = it * tt
        y = None
        for k in range(3):
            s = src[t0 + k:t0 + k + tt, :].astype(_BF)
            if relu:
                s = jnp.maximum(s, jnp.zeros_like(s))
            term = _dot(s, w[k])
            y = term if y is None else y + term
        out[t0:t0 + tt, :] = y + b[:]


def _up2conv3(src, dst, t_in, w, b, ncols=256):
    tt = min(128, t_in)
    for it in range(t_in // tt):
        t0 = it * tt
        s0 = src[t0:t0 + tt, :].astype(_BF)
        s1 = src[t0 + 1:t0 + 1 + tt, :].astype(_BF)
        s2 = src[t0 + 2:t0 + 2 + tt, :].astype(_BF)
        a = _dot(s0, w[0])
        bm = _dot(s1, w[0])
        c = _dot(s1, w[1])
        d = _dot(s1, w[2])
        e = _dot(s2, w[2])
        ye = (a + c) + d + b[:]
        yo = (bm + c) + e + b[:]
        inter = jnp.concatenate([ye, yo], axis=1).reshape(2 * tt, ncols)
        dst[1 + 2 * t0:1 + 2 * t0 + 2 * tt, :] = inter
    _zero_pads(dst, 2 * t_in, ncols)


def _dec_kernel(q_ref,
                wi, bi,
                u0w, u0b, r0w1, r0b1, r0w2, r0b2,
                u1w, u1b, r1w1, r1b1, r1w2, r1b2,
                u2w, u2b, r2w1, r2b1, r2w2, r2b2,
                wo, bo,
                out_ref,
                a, bbuf):
    a[1:65, :] = q_ref[0, :, :]
    _zero_pads(a, 64, 256)
    _conv3(a, bbuf, 64, wi, bi)
    _up2conv3(bbuf, a, 64, u0w, u0b)
    _conv3(a, bbuf, 128, r0w1, r0b1, relu=True)
    _conv3(bbuf, a, 128, r0w2, r0b2, relu=True, accum=True)
    _up2conv3(a, bbuf, 128, u1w, u1b)
    _conv3(bbuf, a, 256, r1w1, r1b1, relu=True)
    _conv3(a, bbuf, 256, r1w2, r1b2, relu=True, accum=True)
    _up2conv3(bbuf, a, 256, u2w, u2b)
    _conv3(a, bbuf, 512, r2w1, r2b1, relu=True)
    _conv3(bbuf, a, 512, r2w2, r2b2, relu=True, accum=True)
    _conv3_out(a, out_ref.at[0], 512, wo, bo, relu=True)


def _full(shape):
    nd = len(shape)
    return pl.BlockSpec(shape, lambda i, _nd=nd: (0,) * _nd)


def kernel(x, params):
    p = params
    bq = x.shape[0]

    feat = _encoder(jnp.transpose(x, (0, 2, 1)), p)
    flat = jnp.transpose(feat, (0, 2, 1)).reshape(-1, 256)
    nrows = flat.shape[0]

    cb = p['codebooks']
    msk = jnp.uint32(0xFFFF0000)
    hi = jax.lax.bitcast_convert_type(
        jax.lax.bitcast_convert_type(cb, jnp.uint32) & msk, _F32)
    rem = cb - hi
    mid = jax.lax.bitcast_convert_type(
        jax.lax.bitcast_convert_type(rem, jnp.uint32) & msk, _F32)
    cb_hi = hi.astype(_BF)
    cb_mid = mid.astype(_BF)
    cb_lo = (rem - mid).astype(_BF)
    cbt = jnp.transpose(cb, (0, 2, 1)).astype(_BF)
    cc = jnp.sum(cb * cb, axis=2)

    def b2(b):
        return b.reshape(1, -1)

    vq_in = [flat, p['vq_w_in'].astype(_BF), b2(p['vq_b_in']),
             cbt, cc, cb_hi, cb_mid, cb_lo,
             p['vq_w_out'].astype(_BF), b2(p['vq_b_out'])]
    vq_specs = [pl.BlockSpec((256, 256), lambda i: (i, 0))]
    vq_specs += [_full(a.shape) for a in vq_in[1:]]

    nblk = nrows // 256
    q_lat, enc_idx, loss_parts = pl.pallas_call(
        _vq_kernel,
        grid=(nblk,),
        in_specs=vq_specs,
        out_specs=[pl.BlockSpec((256, 256), lambda i: (i, 0)),
                   pl.BlockSpec((256, 4), lambda i: (i, 0)),
                   pl.BlockSpec((1, 1, 128), lambda i: (i, 0, 0))],
        out_shape=[jax.ShapeDtypeStruct((nrows, 256), _F32),
                   jax.ShapeDtypeStruct((nrows, 4), jnp.int32),
                   jax.ShapeDtypeStruct((nblk, 1, 128), _F32)],
    )(*vq_in)

    vq_loss = 0.25 * (jnp.sum(loss_parts[:, 0, 0]) / (nrows * 128))

    def tw(w):
        return jnp.transpose(w, (2, 1, 0)).astype(_BF)

    q = q_lat.reshape(bq, 64, 256)
    dec_in = [q]
    dec_specs = [pl.BlockSpec((1, 64, 256), lambda i: (i, 0, 0))]
    dec_in += [tw(p['dec_w_in']), b2(p['dec_b_in'])]
    for i in range(3):
        dec_in += [tw(p['dec_u%d_w' % i]), b2(p['dec_u%d_b' % i]),
                   tw(p['dec_r%d_w1' % i]), b2(p['dec_r%d_b1' % i]),
                   tw(p['dec_r%d_w2' % i]), b2(p['dec_r%d_b2' % i])]
    dec_in += [tw(p['dec_w_out']), b2(p['dec_b_out'])]
    dec_specs += [_full(a.shape) for a in dec_in[1:]]

    x_recon = pl.pallas_call(
        _dec_kernel,
        grid=(bq,),
        in_specs=dec_specs,
        out_specs=pl.BlockSpec((1, 512, 32), lambda i: (i, 0, 0)),
        out_shape=jax.ShapeDtypeStruct((bq, 512, 32), _F32),
        scratch_shapes=[pltpu.VMEM((520, 256), _F32),
                        pltpu.VMEM((520, 256), _F32)],
    )(*dec_in)

    return (x_recon, vq_loss, enc_idx)
